# baseline (speedup 1.0000x reference)
import jax
import jax.numpy as jnp
from jax import lax
from jax.experimental import pallas as pl
from jax.experimental.pallas import tpu as pltpu
from jax.experimental.pallas import tpu_sc as plsc

BATCH = 16384
EMB = 64
NROWS = 1000000
NC = 2
NS = 16
NW = NC * NS
BPW = BATCH // NW
LANES = 16
HALF = BPW // 2
IDXN = HALF * EMB


def _body(p_hbm, q_hbm, uid_hbm, bid_hbm, out_hbm,
          uidx_v, bidx_v, idx_u, idx_q, gat_u, gat_q, out_v, sem_u, sem_q):
    wid = lax.axis_index("s") * NC + lax.axis_index("c")
    base = wid * BPW
    pltpu.sync_copy(uid_hbm.at[pl.ds(base, BPW)], uidx_v)
    pltpu.sync_copy(bid_hbm.at[pl.ds(base, BPW)], bidx_v)

    def half(h, _):
        def build(g, _):
            uvec = uidx_v[pl.ds(h * HALF + g * LANES, LANES)]
            bvec = bidx_v[pl.ds(h * HALF + g * LANES, LANES)]
            for d in range(EMB):
                off = d * NROWS
                idx_u[pl.ds(d * HALF + g * LANES, LANES)] = uvec + off
                idx_q[pl.ds(d * HALF + g * LANES, LANES)] = bvec + off
            return 0
        lax.fori_loop(0, HALF // LANES, build, 0)

        cu = pltpu.async_copy(p_hbm.at[idx_u], gat_u, sem_u)
        cq = pltpu.async_copy(q_hbm.at[idx_q], gat_q, sem_q)
        cu.wait()
        cq.wait()

        def group(g, _):
            acc = jnp.zeros((LANES,), jnp.float32)
            for d in range(EMB):
                u = gat_u[pl.ds(d * HALF + g * LANES, LANES)]
                q = gat_q[pl.ds(d * HALF + g * LANES, LANES)]
                acc = acc + u * q
            out_v[pl.ds(h * HALF + g * LANES, LANES)] = acc
            return 0
        lax.fori_loop(0, HALF // LANES, group, 0)
        return 0

    lax.fori_loop(0, 2, half, 0)
    pltpu.sync_copy(out_v, out_hbm.at[pl.ds(base, BPW)])


_sc_call = pl.kernel(
    _body,
    out_type=jax.ShapeDtypeStruct((BATCH,), jnp.float32),
    mesh=plsc.VectorSubcoreMesh(
        core_axis_name="c", subcore_axis_name="s",
        num_cores=NC, num_subcores=NS),
    scratch_types=[
        pltpu.VMEM((BPW,), jnp.int32),
        pltpu.VMEM((BPW,), jnp.int32),
        pltpu.VMEM((IDXN,), jnp.int32),
        pltpu.VMEM((IDXN,), jnp.int32),
        pltpu.VMEM((IDXN,), jnp.float32),
        pltpu.VMEM((IDXN,), jnp.float32),
        pltpu.VMEM((BPW,), jnp.float32),
        pltpu.SemaphoreType.DMA,
        pltpu.SemaphoreType.DMA,
    ],
    compiler_params=pltpu.CompilerParams(
        needs_layout_passes=False, use_tc_tiling_on_sc=False),
)


@jax.jit
def kernel(P, Q, user_id, book_id):
    pf = P.T.reshape(NROWS * EMB)
    qf = Q.T.reshape(NROWS * EMB)
    return _sc_call(pf, qf,
                    user_id.astype(jnp.int32), book_id.astype(jnp.int32))
